# baseline (speedup 1.0000x reference)
import functools

import jax
import jax.numpy as jnp
from jax import lax
from jax.experimental import pallas as pl
from jax.experimental.pallas import tpu as pltpu
from jax.experimental.pallas import tpu_sc as plsc

_D = 128
_ROW_BLK = 1000
_EDGE_BLK = 80


def _proj_body(x_ref, wpt_ref, bp_ref, out_ref):
    h = jnp.dot(x_ref[...], wpt_ref[...], preferred_element_type=jnp.float32)
    out_ref[...] = jnp.maximum(h + bp_ref[...], 0.0)


def _project(x, wpt, bp):
    n = x.shape[0]
    return pl.pallas_call(
        _proj_body,
        grid=(n // _ROW_BLK,),
        in_specs=[
            pl.BlockSpec((_ROW_BLK, _D), lambda i: (i, 0)),
            pl.BlockSpec((_D, _D), lambda i: (0, 0)),
            pl.BlockSpec((1, _D), lambda i: (0, 0)),
        ],
        out_specs=pl.BlockSpec((_ROW_BLK, _D), lambda i: (i, 0)),
        out_shape=jax.ShapeDtypeStruct((n, _D), jnp.float32),
    )(x, wpt, bp)


def _combine_math(p0, p1, c0, c1, x, wlt, bl, wrt, g, b):
    cnt = jnp.maximum(c0 + c1, 1.0)
    agg = (p0 + p1) / cnt
    out = jnp.dot(agg, wlt, preferred_element_type=jnp.float32)
    out = out + bl
    out = out + jnp.dot(x, wrt, preferred_element_type=jnp.float32)
    mu = jnp.mean(out, axis=1, keepdims=True)
    ctr = out - mu
    var = jnp.mean(ctr * ctr, axis=1, keepdims=True)
    return ctr * lax.rsqrt(var + 1e-5) * g + b


def _combine_body(p0_ref, p1_ref, c0_ref, c1_ref, x_ref, wlt_ref, bl_ref,
                  wrt_ref, g_ref, b_ref, out_ref):
    out_ref[...] = _combine_math(
        p0_ref[...], p1_ref[...], c0_ref[...], c1_ref[...], x_ref[...],
        wlt_ref[...], bl_ref[...], wrt_ref[...], g_ref[...], b_ref[...])


def _fused_body(p0_ref, p1_ref, c0_ref, c1_ref, x_ref, wlt_ref, bl_ref,
                wrt_ref, g_ref, b_ref, wpt_ref, bp_ref, out_ref, hp_ref):
    y = _combine_math(
        p0_ref[...], p1_ref[...], c0_ref[...], c1_ref[...], x_ref[...],
        wlt_ref[...], bl_ref[...], wrt_ref[...], g_ref[...], b_ref[...])
    out_ref[...] = y
    h = jnp.dot(y, wpt_ref[...], preferred_element_type=jnp.float32)
    hp_ref[...] = jnp.maximum(h + bp_ref[...], 0.0)


def _combine_specs(n, npitch):
    full = lambda i: (0, 0)
    row = lambda i: (i, 0)
    shift = lambda i: (npitch // _ROW_BLK + i, 0)
    in_specs = [
        pl.BlockSpec((_ROW_BLK, _D), row),
        pl.BlockSpec((_ROW_BLK, _D), shift),
        pl.BlockSpec((_ROW_BLK, 1), row),
        pl.BlockSpec((_ROW_BLK, 1), shift),
        pl.BlockSpec((_ROW_BLK, _D), row),
        pl.BlockSpec((_D, _D), full),
        pl.BlockSpec((1, _D), full),
        pl.BlockSpec((_D, _D), full),
        pl.BlockSpec((1, _D), full),
        pl.BlockSpec((1, _D), full),
    ]
    return in_specs, row, full


def _combine(parts, cnts2, x, wlt, bl, wrt, g, b, npitch):
    n = x.shape[0]
    in_specs, row, full = _combine_specs(n, npitch)
    return pl.pallas_call(
        _combine_body,
        grid=(n // _ROW_BLK,),
        in_specs=in_specs,
        out_specs=pl.BlockSpec((_ROW_BLK, _D), row),
        out_shape=jax.ShapeDtypeStruct((n, _D), jnp.float32),
    )(parts, parts, cnts2, cnts2, x, wlt, bl, wrt, g, b)


def _fused_combine_project(parts, cnts2, x, wlt, bl, wrt, g, b, wpt, bp,
                           npitch):
    n = x.shape[0]
    in_specs, row, full = _combine_specs(n, npitch)
    in_specs += [pl.BlockSpec((_D, _D), full), pl.BlockSpec((1, _D), full)]
    return pl.pallas_call(
        _fused_body,
        grid=(n // _ROW_BLK,),
        in_specs=in_specs,
        out_specs=[pl.BlockSpec((_ROW_BLK, _D), row),
                   pl.BlockSpec((_ROW_BLK, _D), row)],
        out_shape=[jax.ShapeDtypeStruct((n, _D), jnp.float32),
                   jax.ShapeDtypeStruct((n, _D), jnp.float32)],
    )(parts, parts, cnts2, cnts2, x, wlt, bl, wrt, g, b, wpt, bp)


@functools.lru_cache(maxsize=None)
def _sc_geom(n, e):
    mesh = plsc.VectorSubcoreMesh(core_axis_name="c", subcore_axis_name="s")
    nc, ns = mesh.num_cores, mesh.num_subcores
    rpt = -(-n // (8 * ns)) * 8
    npad = rpt * ns
    npitch = -(-npad // _ROW_BLK) * _ROW_BLK
    return mesh, nc, ns, rpt, npad, npitch, e // (nc * ns)


_BLK = 125


@functools.lru_cache(maxsize=None)
def _make_segsum(n, e, with_degree):
    mesh, nc, ns, rpt, npad, npitch, epw = _sc_geom(n, e)
    nblk = epw // _BLK

    half = nblk // 2

    out_type = [jax.ShapeDtypeStruct((nc * npitch, _D), jnp.float32)]
    scratch = [
        pltpu.VMEM((half, _BLK), jnp.int32),
        pltpu.VMEM((half, _BLK), jnp.int32),
        pltpu.VMEM((_BLK, _D), jnp.float32),
        pltpu.VMEM((_BLK, _D), jnp.float32),
        pltpu.VMEM_SHARED((npad, _D), jnp.float32),
        pltpu.SemaphoreType.DMA,
        pltpu.SemaphoreType.DMA,
        pltpu.SemaphoreType.DMA,
        pltpu.SemaphoreType.DMA,
    ]
    if with_degree:
        out_type.append(jax.ShapeDtypeStruct((nc * npitch,), jnp.float32))
        scratch.append(pltpu.VMEM((_BLK,), jnp.float32))
        scratch.append(pltpu.VMEM((rpt,), jnp.float32))
        scratch.append(pltpu.VMEM_SHARED((npad,), jnp.float32))
    else:
        out_type = out_type[0]

    @functools.partial(pl.kernel, out_type=out_type, mesh=mesh,
                       scratch_types=scratch)
    def segsum(h_hbm, src_hbm, dst_hbm, zeros_hbm, zeros1_hbm, ones_hbm,
               *rest):
        if with_degree:
            (out_hbm, cnt_hbm, sidx_v, didx_v, rows0_v, rows1_v, agg_sh,
             gs0, gs1, ss0, ss1, ones_v, cstage_v, cnt_sh) = rest
        else:
            (out_hbm, sidx_v, didx_v, rows0_v, rows1_v, agg_sh,
             gs0, gs1, ss0, ss1) = rest
        c = lax.axis_index("c")
        s = lax.axis_index("s")
        wid = c * ns + s

        pltpu.sync_copy(zeros_hbm.at[pl.ds(s * rpt, rpt)],
                        agg_sh.at[pl.ds(s * rpt, rpt)])
        if with_degree:
            pltpu.sync_copy(ones_hbm, ones_v)
            pltpu.sync_copy(zeros1_hbm.at[pl.ds(s * rpt, rpt)], cstage_v)
            pltpu.sync_copy(cstage_v, cnt_sh.at[pl.ds(s * rpt, rpt)])
        plsc.subcore_barrier()

        def block(i, didx_ref, rows_ref, ssem):
            pltpu.async_copy(rows_ref, agg_sh.at[didx_ref.at[i]], ssem,
                             add=True)
            if with_degree:
                pltpu.sync_copy(ones_v, cnt_sh.at[didx_ref.at[i]], add=True)

        for hf in range(2):
            base = wid * nblk + hf * half
            pltpu.sync_copy(src_hbm.at[pl.ds(base, half)], sidx_v)
            pltpu.sync_copy(dst_hbm.at[pl.ds(base, half)], didx_v)

            pltpu.async_copy(h_hbm.at[sidx_v.at[0]], rows0_v, gs0)
            pltpu.async_copy(h_hbm.at[sidx_v.at[1]], rows1_v, gs1)

            def body(j, carry):
                i = 2 * j
                pltpu.make_async_copy(
                    h_hbm.at[sidx_v.at[i]], rows0_v, gs0).wait()
                block(i, didx_v, rows0_v, ss0)
                pltpu.make_async_copy(
                    h_hbm.at[sidx_v.at[i + 1]], rows1_v, gs1).wait()
                block(i + 1, didx_v, rows1_v, ss1)
                pltpu.make_async_copy(
                    rows0_v, agg_sh.at[didx_v.at[i]], ss0).wait()

                @pl.when(j + 1 < half // 2)
                def _():
                    pltpu.async_copy(
                        h_hbm.at[sidx_v.at[i + 2]], rows0_v, gs0)

                pltpu.make_async_copy(
                    rows1_v, agg_sh.at[didx_v.at[i + 1]], ss1).wait()

                @pl.when(j + 1 < half // 2)
                def _():
                    pltpu.async_copy(
                        h_hbm.at[sidx_v.at[i + 3]], rows1_v, gs1)

                return carry

            lax.fori_loop(0, half // 2, body, 0)

        plsc.subcore_barrier()

        pltpu.sync_copy(agg_sh.at[pl.ds(s * rpt, rpt)],
                        out_hbm.at[pl.ds(c * npitch + s * rpt, rpt)])
        if with_degree:
            pltpu.sync_copy(cnt_sh.at[pl.ds(s * rpt, rpt)], cstage_v)
            pltpu.sync_copy(cstage_v,
                            cnt_hbm.at[pl.ds(c * npitch + s * rpt, rpt)])

    return segsum


def kernel(x, edge_index, Wp0, bp0, Wl0, bl0, Wr0, g0, b0,
           Wp1, bp1, Wl1, bl1, Wr1, g1, b1):
    n = x.shape[0]
    e = edge_index.shape[1]
    src = edge_index[0]
    dst = edge_index[1]
    src2 = src.reshape(e // _BLK, _BLK)
    dst2 = dst.reshape(e // _BLK, _BLK)
    _, nc, ns, rpt, npad, npitch, _ = _sc_geom(n, e)
    segsum0 = _make_segsum(n, e, True)
    segsum1 = _make_segsum(n, e, False)
    zeros = jnp.zeros((npad, _D), jnp.float32)
    zeros1 = jnp.zeros((npad,), jnp.float32)
    ones = jnp.ones((_BLK,), jnp.float32)

    hp0 = _project(x, Wp0.T, bp0.reshape(1, _D))
    parts0, cnts = segsum0(hp0, src2, dst2, zeros, zeros1, ones)
    cnts2 = cnts.reshape(nc * npitch, 1)
    h0, hp1 = _fused_combine_project(
        parts0, cnts2, x, Wl0.T, bl0.reshape(1, _D), Wr0.T,
        g0.reshape(1, _D), b0.reshape(1, _D), Wp1.T, bp1.reshape(1, _D),
        npitch)
    parts1 = segsum1(hp1, src2, dst2, zeros, zeros1, ones)
    h1 = _combine(parts1, cnts2, h0, Wl1.T, bl1.reshape(1, _D), Wr1.T,
                  g1.reshape(1, _D), b1.reshape(1, _D), npitch)
    return h1


# speedup vs baseline: 1.2273x; 1.2273x over previous
import functools

import jax
import jax.numpy as jnp
from jax import lax
from jax.experimental import pallas as pl
from jax.experimental.pallas import tpu as pltpu
from jax.experimental.pallas import tpu_sc as plsc

_D = 128
_ROW_BLK = 1000
_EDGE_BLK = 80


def _proj_body(x_ref, wpt_ref, bp_ref, out_ref):
    h = jnp.dot(x_ref[...], wpt_ref[...], preferred_element_type=jnp.float32)
    out_ref[...] = jnp.maximum(h + bp_ref[...], 0.0)


def _project(x, wpt, bp):
    n = x.shape[0]
    return pl.pallas_call(
        _proj_body,
        grid=(n // _ROW_BLK,),
        in_specs=[
            pl.BlockSpec((_ROW_BLK, _D), lambda i: (i, 0)),
            pl.BlockSpec((_D, _D), lambda i: (0, 0)),
            pl.BlockSpec((1, _D), lambda i: (0, 0)),
        ],
        out_specs=pl.BlockSpec((_ROW_BLK, _D), lambda i: (i, 0)),
        out_shape=jax.ShapeDtypeStruct((n, _D), jnp.float32),
    )(x, wpt, bp)


def _combine_math(p0, p1, c0, c1, x, wlt, bl, wrt, g, b):
    cnt = jnp.maximum(c0 + c1, 1.0)
    agg = (p0 + p1) / cnt
    out = jnp.dot(agg, wlt, preferred_element_type=jnp.float32)
    out = out + bl
    out = out + jnp.dot(x, wrt, preferred_element_type=jnp.float32)
    mu = jnp.mean(out, axis=1, keepdims=True)
    ctr = out - mu
    var = jnp.mean(ctr * ctr, axis=1, keepdims=True)
    return ctr * lax.rsqrt(var + 1e-5) * g + b


def _combine_body(p0_ref, p1_ref, c0_ref, c1_ref, x_ref, wlt_ref, bl_ref,
                  wrt_ref, g_ref, b_ref, out_ref):
    out_ref[...] = _combine_math(
        p0_ref[...], p1_ref[...], c0_ref[...], c1_ref[...], x_ref[...],
        wlt_ref[...], bl_ref[...], wrt_ref[...], g_ref[...], b_ref[...])


def _fused_body(p0_ref, p1_ref, c0_ref, c1_ref, x_ref, wlt_ref, bl_ref,
                wrt_ref, g_ref, b_ref, wpt_ref, bp_ref, out_ref, hp_ref):
    y = _combine_math(
        p0_ref[...], p1_ref[...], c0_ref[...], c1_ref[...], x_ref[...],
        wlt_ref[...], bl_ref[...], wrt_ref[...], g_ref[...], b_ref[...])
    out_ref[...] = y
    h = jnp.dot(y, wpt_ref[...], preferred_element_type=jnp.float32)
    hp_ref[...] = jnp.maximum(h + bp_ref[...], 0.0)


def _combine_specs(n, npitch):
    full = lambda i: (0, 0)
    row = lambda i: (i, 0)
    shift = lambda i: (npitch // _ROW_BLK + i, 0)
    in_specs = [
        pl.BlockSpec((_ROW_BLK, _D), row),
        pl.BlockSpec((_ROW_BLK, _D), shift),
        pl.BlockSpec((_ROW_BLK, 1), row),
        pl.BlockSpec((_ROW_BLK, 1), shift),
        pl.BlockSpec((_ROW_BLK, _D), row),
        pl.BlockSpec((_D, _D), full),
        pl.BlockSpec((1, _D), full),
        pl.BlockSpec((_D, _D), full),
        pl.BlockSpec((1, _D), full),
        pl.BlockSpec((1, _D), full),
    ]
    return in_specs, row, full


def _combine(parts, cnts2, x, wlt, bl, wrt, g, b, npitch):
    n = x.shape[0]
    in_specs, row, full = _combine_specs(n, npitch)
    return pl.pallas_call(
        _combine_body,
        grid=(n // _ROW_BLK,),
        in_specs=in_specs,
        out_specs=pl.BlockSpec((_ROW_BLK, _D), row),
        out_shape=jax.ShapeDtypeStruct((n, _D), jnp.float32),
    )(parts, parts, cnts2, cnts2, x, wlt, bl, wrt, g, b)


def _fused_combine_project(parts, cnts2, x, wlt, bl, wrt, g, b, wpt, bp,
                           npitch):
    n = x.shape[0]
    in_specs, row, full = _combine_specs(n, npitch)
    in_specs += [pl.BlockSpec((_D, _D), full), pl.BlockSpec((1, _D), full)]
    return pl.pallas_call(
        _fused_body,
        grid=(n // _ROW_BLK,),
        in_specs=in_specs,
        out_specs=[pl.BlockSpec((_ROW_BLK, _D), row),
                   pl.BlockSpec((_ROW_BLK, _D), row)],
        out_shape=[jax.ShapeDtypeStruct((n, _D), jnp.float32),
                   jax.ShapeDtypeStruct((n, _D), jnp.float32)],
    )(parts, parts, cnts2, cnts2, x, wlt, bl, wrt, g, b, wpt, bp)


@functools.lru_cache(maxsize=None)
def _sc_geom(n, e):
    mesh = plsc.VectorSubcoreMesh(core_axis_name="c", subcore_axis_name="s")
    nc, ns = mesh.num_cores, mesh.num_subcores
    rpt = -(-n // (8 * ns)) * 8
    npad = rpt * ns
    npitch = -(-npad // _ROW_BLK) * _ROW_BLK
    return mesh, nc, ns, rpt, npad, npitch, e // (nc * ns)


_BLK = 125


@functools.lru_cache(maxsize=None)
def _make_segsum(n, e, with_degree):
    mesh, nc, ns, rpt, npad, npitch, epw = _sc_geom(n, e)
    nblk = epw // _BLK

    half = nblk // 2

    out_type = [jax.ShapeDtypeStruct((nc * npitch, _D), jnp.float32)]
    scratch = [
        pltpu.VMEM((half, _BLK), jnp.int32),
        pltpu.VMEM((half, _BLK), jnp.int32),
        pltpu.VMEM((_BLK, _D), jnp.float32),
        pltpu.VMEM((_BLK, _D), jnp.float32),
        pltpu.VMEM_SHARED((npad, _D), jnp.float32),
        pltpu.SemaphoreType.DMA,
        pltpu.SemaphoreType.DMA,
        pltpu.SemaphoreType.DMA,
        pltpu.SemaphoreType.DMA,
    ]
    if with_degree:
        out_type.append(jax.ShapeDtypeStruct((nc * npitch,), jnp.float32))
        scratch.append(pltpu.VMEM((_BLK,), jnp.float32))
        scratch.append(pltpu.VMEM((rpt,), jnp.float32))
        scratch.append(pltpu.VMEM_SHARED((npad,), jnp.float32))
    else:
        out_type = out_type[0]

    @functools.partial(pl.kernel, out_type=out_type, mesh=mesh,
                       scratch_types=scratch)
    def segsum(h_hbm, src_hbm, dst_hbm, zeros_hbm, zeros1_hbm, ones_hbm,
               *rest):
        if with_degree:
            (out_hbm, cnt_hbm, sidx_v, didx_v, rows0_v, rows1_v, agg_sh,
             gs0, gs1, ss0, ss1, ones_v, cstage_v, cnt_sh) = rest
        else:
            (out_hbm, sidx_v, didx_v, rows0_v, rows1_v, agg_sh,
             gs0, gs1, ss0, ss1) = rest
        c = lax.axis_index("c")
        s = lax.axis_index("s")
        wid = c * ns + s

        pltpu.sync_copy(zeros_hbm.at[pl.ds(s * rpt, rpt)],
                        agg_sh.at[pl.ds(s * rpt, rpt)])
        if with_degree:
            pltpu.sync_copy(ones_hbm, ones_v)
            pltpu.sync_copy(zeros1_hbm.at[pl.ds(s * rpt, rpt)], cstage_v)
            pltpu.sync_copy(cstage_v, cnt_sh.at[pl.ds(s * rpt, rpt)])
        plsc.subcore_barrier()

        def block(i, didx_ref, rows_ref, ssem):
            del ssem
            pltpu.sync_copy(rows_ref, agg_sh.at[didx_ref.at[i]], add=True)
            if with_degree:
                pltpu.sync_copy(ones_v, cnt_sh.at[didx_ref.at[i]], add=True)

        for hf in range(2):
            base = wid * nblk + hf * half
            pltpu.sync_copy(src_hbm.at[pl.ds(base, half)], sidx_v)
            pltpu.sync_copy(dst_hbm.at[pl.ds(base, half)], didx_v)

            pltpu.async_copy(h_hbm.at[sidx_v.at[0]], rows0_v, gs0)
            pltpu.async_copy(h_hbm.at[sidx_v.at[1]], rows1_v, gs1)

            def body(j, carry):
                i = 2 * j
                pltpu.make_async_copy(
                    h_hbm.at[sidx_v.at[i]], rows0_v, gs0).wait()
                block(i, didx_v, rows0_v, ss0)

                @pl.when(j + 1 < half // 2)
                def _():
                    pltpu.async_copy(
                        h_hbm.at[sidx_v.at[i + 2]], rows0_v, gs0)

                pltpu.make_async_copy(
                    h_hbm.at[sidx_v.at[i + 1]], rows1_v, gs1).wait()
                block(i + 1, didx_v, rows1_v, ss1)

                @pl.when(j + 1 < half // 2)
                def _():
                    pltpu.async_copy(
                        h_hbm.at[sidx_v.at[i + 3]], rows1_v, gs1)

                return carry

            lax.fori_loop(0, half // 2, body, 0)

        plsc.subcore_barrier()

        pltpu.sync_copy(agg_sh.at[pl.ds(s * rpt, rpt)],
                        out_hbm.at[pl.ds(c * npitch + s * rpt, rpt)])
        if with_degree:
            pltpu.sync_copy(cnt_sh.at[pl.ds(s * rpt, rpt)], cstage_v)
            pltpu.sync_copy(cstage_v,
                            cnt_hbm.at[pl.ds(c * npitch + s * rpt, rpt)])

    return segsum


def kernel(x, edge_index, Wp0, bp0, Wl0, bl0, Wr0, g0, b0,
           Wp1, bp1, Wl1, bl1, Wr1, g1, b1):
    n = x.shape[0]
    e = edge_index.shape[1]
    src = edge_index[0]
    dst = edge_index[1]
    src2 = src.reshape(e // _BLK, _BLK)
    dst2 = dst.reshape(e // _BLK, _BLK)
    _, nc, ns, rpt, npad, npitch, _ = _sc_geom(n, e)
    segsum0 = _make_segsum(n, e, True)
    segsum1 = _make_segsum(n, e, False)
    zeros = jnp.zeros((npad, _D), jnp.float32)
    zeros1 = jnp.zeros((npad,), jnp.float32)
    ones = jnp.ones((_BLK,), jnp.float32)

    hp0 = _project(x, Wp0.T, bp0.reshape(1, _D))
    parts0, cnts = segsum0(hp0, src2, dst2, zeros, zeros1, ones)
    cnts2 = cnts.reshape(nc * npitch, 1)
    h0, hp1 = _fused_combine_project(
        parts0, cnts2, x, Wl0.T, bl0.reshape(1, _D), Wr0.T,
        g0.reshape(1, _D), b0.reshape(1, _D), Wp1.T, bp1.reshape(1, _D),
        npitch)
    parts1 = segsum1(hp1, src2, dst2, zeros, zeros1, ones)
    h1 = _combine(parts1, cnts2, h0, Wl1.T, bl1.reshape(1, _D), Wr1.T,
                  g1.reshape(1, _D), b1.reshape(1, _D), npitch)
    return h1


# speedup vs baseline: 1.2281x; 1.0007x over previous
import functools

import jax
import jax.numpy as jnp
from jax import lax
from jax.experimental import pallas as pl
from jax.experimental.pallas import tpu as pltpu
from jax.experimental.pallas import tpu_sc as plsc

_D = 128
_ROW_BLK = 1000


def _proj_body(x_ref, wpt_ref, bp_ref, out_ref):
    h = jnp.dot(x_ref[...], wpt_ref[...], preferred_element_type=jnp.float32)
    out_ref[...] = jnp.maximum(h + bp_ref[...], 0.0)


def _project(x, wpt, bp):
    n = x.shape[0]
    return pl.pallas_call(
        _proj_body,
        grid=(n // _ROW_BLK,),
        in_specs=[
            pl.BlockSpec((_ROW_BLK, _D), lambda i: (i, 0)),
            pl.BlockSpec((_D, _D), lambda i: (0, 0)),
            pl.BlockSpec((1, _D), lambda i: (0, 0)),
        ],
        out_specs=pl.BlockSpec((_ROW_BLK, _D), lambda i: (i, 0)),
        out_shape=jax.ShapeDtypeStruct((n, _D), jnp.float32),
    )(x, wpt, bp)


def _combine_math(p0, p1, c0, c1, x, wlt, bl, wrt, g, b):
    cnt = jnp.maximum(c0 + c1, 1.0)
    agg = (p0 + p1) / cnt
    out = jnp.dot(agg, wlt, preferred_element_type=jnp.float32)
    out = out + bl
    out = out + jnp.dot(x, wrt, preferred_element_type=jnp.float32)
    mu = jnp.mean(out, axis=1, keepdims=True)
    ctr = out - mu
    var = jnp.mean(ctr * ctr, axis=1, keepdims=True)
    return ctr * lax.rsqrt(var + 1e-5) * g + b


def _combine_body(p0_ref, p1_ref, c0_ref, c1_ref, x_ref, wlt_ref, bl_ref,
                  wrt_ref, g_ref, b_ref, out_ref):
    out_ref[...] = _combine_math(
        p0_ref[...], p1_ref[...], c0_ref[...], c1_ref[...], x_ref[...],
        wlt_ref[...], bl_ref[...], wrt_ref[...], g_ref[...], b_ref[...])


def _fused_body(p0_ref, p1_ref, c0_ref, c1_ref, x_ref, wlt_ref, bl_ref,
                wrt_ref, g_ref, b_ref, wpt_ref, bp_ref, out_ref, hp_ref):
    y = _combine_math(
        p0_ref[...], p1_ref[...], c0_ref[...], c1_ref[...], x_ref[...],
        wlt_ref[...], bl_ref[...], wrt_ref[...], g_ref[...], b_ref[...])
    out_ref[...] = y
    h = jnp.dot(y, wpt_ref[...], preferred_element_type=jnp.float32)
    hp_ref[...] = jnp.maximum(h + bp_ref[...], 0.0)


def _combine_specs(n, npitch):
    full = lambda i: (0, 0)
    row = lambda i: (i, 0)
    shift = lambda i: (npitch // _ROW_BLK + i, 0)
    in_specs = [
        pl.BlockSpec((_ROW_BLK, _D), row),
        pl.BlockSpec((_ROW_BLK, _D), shift),
        pl.BlockSpec((_ROW_BLK, 1), row),
        pl.BlockSpec((_ROW_BLK, 1), shift),
        pl.BlockSpec((_ROW_BLK, _D), row),
        pl.BlockSpec((_D, _D), full),
        pl.BlockSpec((1, _D), full),
        pl.BlockSpec((_D, _D), full),
        pl.BlockSpec((1, _D), full),
        pl.BlockSpec((1, _D), full),
    ]
    return in_specs, row, full


def _combine(parts, cnts2, x, wlt, bl, wrt, g, b, npitch):
    n = x.shape[0]
    in_specs, row, full = _combine_specs(n, npitch)
    return pl.pallas_call(
        _combine_body,
        grid=(n // _ROW_BLK,),
        in_specs=in_specs,
        out_specs=pl.BlockSpec((_ROW_BLK, _D), row),
        out_shape=jax.ShapeDtypeStruct((n, _D), jnp.float32),
    )(parts, parts, cnts2, cnts2, x, wlt, bl, wrt, g, b)


def _fused_combine_project(parts, cnts2, x, wlt, bl, wrt, g, b, wpt, bp,
                           npitch):
    n = x.shape[0]
    in_specs, row, full = _combine_specs(n, npitch)
    in_specs += [pl.BlockSpec((_D, _D), full), pl.BlockSpec((1, _D), full)]
    return pl.pallas_call(
        _fused_body,
        grid=(n // _ROW_BLK,),
        in_specs=in_specs,
        out_specs=[pl.BlockSpec((_ROW_BLK, _D), row),
                   pl.BlockSpec((_ROW_BLK, _D), row)],
        out_shape=[jax.ShapeDtypeStruct((n, _D), jnp.float32),
                   jax.ShapeDtypeStruct((n, _D), jnp.float32)],
    )(parts, parts, cnts2, cnts2, x, wlt, bl, wrt, g, b, wpt, bp)


@functools.lru_cache(maxsize=None)
def _sc_geom(n, e):
    mesh = plsc.VectorSubcoreMesh(core_axis_name="c", subcore_axis_name="s")
    nc, ns = mesh.num_cores, mesh.num_subcores
    rpt = -(-n // (8 * ns)) * 8
    npad = rpt * ns
    npitch = -(-npad // _ROW_BLK) * _ROW_BLK
    return mesh, nc, ns, rpt, npad, npitch, e // (nc * ns)


_BLK = 125


@functools.lru_cache(maxsize=None)
def _make_segsum(n, e, with_degree):
    mesh, nc, ns, rpt, npad, npitch, epw = _sc_geom(n, e)
    nblk = epw // _BLK

    half = nblk // 2

    out_type = [jax.ShapeDtypeStruct((nc * npitch, _D), jnp.float32)]
    scratch = [
        pltpu.VMEM((half, _BLK), jnp.int32),
        pltpu.VMEM((half, _BLK), jnp.int32),
        pltpu.VMEM((_BLK, _D), jnp.float32),
        pltpu.VMEM((_BLK, _D), jnp.float32),
        pltpu.VMEM_SHARED((npad, _D), jnp.float32),
        pltpu.SemaphoreType.DMA,
        pltpu.SemaphoreType.DMA,
    ]
    if with_degree:
        out_type.append(jax.ShapeDtypeStruct((nc * npitch,), jnp.float32))
        scratch.append(pltpu.VMEM((_BLK,), jnp.float32))
        scratch.append(pltpu.VMEM((rpt,), jnp.float32))
        scratch.append(pltpu.VMEM_SHARED((npad,), jnp.float32))
    else:
        out_type = out_type[0]

    @functools.partial(pl.kernel, out_type=out_type, mesh=mesh,
                       scratch_types=scratch)
    def segsum(h_hbm, src_hbm, dst_hbm, zeros_hbm, zeros1_hbm, ones_hbm,
               *rest):
        if with_degree:
            (out_hbm, cnt_hbm, sidx_v, didx_v, rows0_v, rows1_v, agg_sh,
             gs0, gs1, ones_v, cstage_v, cnt_sh) = rest
        else:
            (out_hbm, sidx_v, didx_v, rows0_v, rows1_v, agg_sh,
             gs0, gs1) = rest
        c = lax.axis_index("c")
        s = lax.axis_index("s")
        wid = c * ns + s

        pltpu.sync_copy(zeros_hbm.at[pl.ds(s * rpt, rpt)],
                        agg_sh.at[pl.ds(s * rpt, rpt)])
        if with_degree:
            pltpu.sync_copy(ones_hbm, ones_v)
            pltpu.sync_copy(zeros1_hbm.at[pl.ds(s * rpt, rpt)], cstage_v)
            pltpu.sync_copy(cstage_v, cnt_sh.at[pl.ds(s * rpt, rpt)])
        plsc.subcore_barrier()

        def block(i, didx_ref, rows_ref):
            pltpu.sync_copy(rows_ref, agg_sh.at[didx_ref.at[i]], add=True)
            if with_degree:
                pltpu.sync_copy(ones_v, cnt_sh.at[didx_ref.at[i]], add=True)

        for hf in range(2):
            base = wid * nblk + hf * half
            pltpu.sync_copy(src_hbm.at[pl.ds(base, half)], sidx_v)
            pltpu.sync_copy(dst_hbm.at[pl.ds(base, half)], didx_v)

            pltpu.async_copy(h_hbm.at[sidx_v.at[0]], rows0_v, gs0)
            pltpu.async_copy(h_hbm.at[sidx_v.at[1]], rows1_v, gs1)

            def body(j, carry):
                i = 2 * j
                pltpu.make_async_copy(
                    h_hbm.at[sidx_v.at[i]], rows0_v, gs0).wait()
                block(i, didx_v, rows0_v)

                @pl.when(j + 1 < half // 2)
                def _():
                    pltpu.async_copy(
                        h_hbm.at[sidx_v.at[i + 2]], rows0_v, gs0)

                pltpu.make_async_copy(
                    h_hbm.at[sidx_v.at[i + 1]], rows1_v, gs1).wait()
                block(i + 1, didx_v, rows1_v)

                @pl.when(j + 1 < half // 2)
                def _():
                    pltpu.async_copy(
                        h_hbm.at[sidx_v.at[i + 3]], rows1_v, gs1)

                return carry

            lax.fori_loop(0, half // 2, body, 0)

        plsc.subcore_barrier()

        pltpu.sync_copy(agg_sh.at[pl.ds(s * rpt, rpt)],
                        out_hbm.at[pl.ds(c * npitch + s * rpt, rpt)])
        if with_degree:
            pltpu.sync_copy(cnt_sh.at[pl.ds(s * rpt, rpt)], cstage_v)
            pltpu.sync_copy(cstage_v,
                            cnt_hbm.at[pl.ds(c * npitch + s * rpt, rpt)])

    return segsum


def kernel(x, edge_index, Wp0, bp0, Wl0, bl0, Wr0, g0, b0,
           Wp1, bp1, Wl1, bl1, Wr1, g1, b1):
    n = x.shape[0]
    e = edge_index.shape[1]
    src = edge_index[0]
    dst = edge_index[1]
    src2 = src.reshape(e // _BLK, _BLK)
    dst2 = dst.reshape(e // _BLK, _BLK)
    _, nc, ns, rpt, npad, npitch, _ = _sc_geom(n, e)
    segsum0 = _make_segsum(n, e, True)
    segsum1 = _make_segsum(n, e, False)
    zeros = jnp.zeros((npad, _D), jnp.float32)
    zeros1 = jnp.zeros((npad,), jnp.float32)
    ones = jnp.ones((_BLK,), jnp.float32)

    hp0 = _project(x, Wp0.T, bp0.reshape(1, _D))
    parts0, cnts = segsum0(hp0, src2, dst2, zeros, zeros1, ones)
    cnts2 = cnts.reshape(nc * npitch, 1)
    h0, hp1 = _fused_combine_project(
        parts0, cnts2, x, Wl0.T, bl0.reshape(1, _D), Wr0.T,
        g0.reshape(1, _D), b0.reshape(1, _D), Wp1.T, bp1.reshape(1, _D),
        npitch)
    parts1 = segsum1(hp1, src2, dst2, zeros, zeros1, ones)
    h1 = _combine(parts1, cnts2, h0, Wl1.T, bl1.reshape(1, _D), Wr1.T,
                  g1.reshape(1, _D), b1.reshape(1, _D), npitch)
    return h1
